# baseline (speedup 1.0000x reference)
import jax
import jax.numpy as jnp
from jax.experimental import pallas as pl
from jax.experimental.pallas import tpu as pltpu

N = 10000
D = 128
HID = 32
TOL = 1e-5
TI1 = 256
TI2 = 512
F32 = jnp.float32
BF16 = jnp.bfloat16
HIGHEST = jax.lax.Precision.HIGHEST
_PARAMS = pltpu.CompilerParams(vmem_limit_bytes=63 * 1024 * 1024)


def _self_loop_add(diag_blk, ti):
    rows = jax.lax.broadcasted_iota(jnp.int32, (ti, ti), 0)
    cols = jax.lax.broadcasted_iota(jnp.int32, (ti, ti), 1)
    dvals = jnp.sum(jnp.where(rows == cols, diag_blk, 0.0), axis=1,
                    keepdims=True)
    return jnp.where(jnp.abs(dvals) < TOL, 1.0, 0.0)


def _row_mask(i, ti):
    row_ids = i * ti + jax.lax.broadcasted_iota(jnp.int32, (ti, 1), 0)
    return row_ids < N


def _split_cat(h):
    hi = h.astype(BF16)
    lo = (h - hi.astype(F32)).astype(BF16)
    return jnp.concatenate([hi, lo], axis=1)


def _tanh(x):
    a1 = 4.89352455891786e-03
    a3 = 6.37261928875436e-04
    a5 = 1.48572235717979e-05
    a7 = 5.12229709037114e-08
    a9 = -8.60467152213735e-11
    a11 = 2.00018790482477e-13
    a13 = -2.76076847742355e-16
    b0 = 4.89352518554385e-03
    b2 = 2.26843463243900e-03
    b4 = 1.18534705686654e-04
    b6 = 1.19825839466702e-06
    t = jnp.clip(x, -7.90531110763549805, 7.90531110763549805)
    s = t * t
    p = a11 + s * a13
    p = a9 + s * p
    p = a7 + s * p
    p = a5 + s * p
    p = a3 + s * p
    p = a1 + s * p
    p = t * p
    q = b4 + s * b6
    q = b2 + s * q
    q = b0 + s * q
    y = p / q
    return jnp.where(jnp.abs(x) < 0.0004, x, y)


def _dot_pair(u, wcat, hid):
    u_hi = u.astype(BF16)
    u_lo = (u - u_hi.astype(F32)).astype(BF16)
    dsum = (jnp.dot(u_hi, wcat, preferred_element_type=F32)
            + jnp.dot(u_lo, wcat, preferred_element_type=F32))
    return dsum[:, :hid] + dsum[:, hid:]


def _rsqrt(m):
    s = jax.lax.rsqrt(m)
    s = s * (1.5 - 0.5 * m * s * s)
    s = s * (1.5 - 0.5 * m * s * s)
    return s


def _sage_update(h_self, aggm, wcat, b, din):
    u = jnp.concatenate([h_self, aggm], axis=1)
    h = _dot_pair(u, wcat, HID) + b
    sq = jnp.sum(h * h, axis=-1, keepdims=True)
    h = h * _rsqrt(jnp.maximum(sq, 1e-12))
    return _tanh(h)


def _layer1_body(a_ref, adiag_ref, xcat_ref, xs_ref, w_ref, b_ref,
                 h_ref, hcat_ref, sh_ref, deg_ref, add_ref, r_ref, rs_ref,
                 sx_acc, sh_acc):
    i = pl.program_id(0)
    x_self = xs_ref[...]
    add = _self_loop_add(adiag_ref[...], TI1)

    t = a_ref[...] - 0.5
    r16 = t.astype(BF16)
    r_ref[...] = r16
    tl = t - r16.astype(F32)
    r_lo = tl.astype(BF16)
    rs = jnp.sum(tl, axis=1, keepdims=True)

    @pl.when(i == 0)
    def _init():
        xcat = xcat_ref[...]
        x_full = xcat[:, :D].astype(F32) + xcat[:, D:2 * D].astype(F32)
        sx_acc[...] = jnp.sum(x_full, axis=0, keepdims=True)
        sh_acc[...] = jnp.zeros_like(sh_acc)

    dot = (jnp.dot(r16, xcat_ref[...], preferred_element_type=F32)
           + jnp.dot(r_lo, xcat_ref[...], preferred_element_type=F32))
    agg = dot[:, :D] + dot[:, D:2 * D]
    deg = jnp.sum(a_ref[...], axis=1, keepdims=True) + add
    agg = agg + 0.5 * sx_acc[...] + add * x_self
    h = _sage_update(x_self, agg / deg, w_ref[...], b_ref[...], D)

    h_ref[...] = h
    hcat_ref[...] = _split_cat(h)
    deg_ref[...] = deg
    add_ref[...] = add
    rs_ref[...] = rs

    sh_acc[...] += jnp.sum(jnp.where(_row_mask(i, TI1), h, 0.0), axis=0,
                           keepdims=True)

    @pl.when(i == pl.num_programs(0) - 1)
    def _emit_sh():
        sh_ref[...] = sh_acc[...]


def _layer2_body(r_ref, hcat_ref, hs_ref, sh_ref, deg_ref, add_ref, rs_ref,
                 w_ref, b_ref, o_ref, ocat_ref, so_ref, so_acc):
    i = pl.program_id(0)
    h_self = hs_ref[...]

    @pl.when(i == 0)
    def _init():
        so_acc[...] = jnp.zeros_like(so_acc)

    dot = jnp.dot(r_ref[...], hcat_ref[...], preferred_element_type=F32)
    agg = dot[:, :HID] + dot[:, HID:]
    agg = agg + (0.5 + rs_ref[...] * (1.0 / N)) * sh_ref[...] \
        + add_ref[...] * h_self
    h = _sage_update(h_self, agg / deg_ref[...], w_ref[...], b_ref[...], HID)

    o_ref[...] = h
    ocat_ref[...] = _split_cat(h)
    so_acc[...] += jnp.sum(jnp.where(_row_mask(i, TI2), h, 0.0), axis=0,
                           keepdims=True)

    @pl.when(i == pl.num_programs(0) - 1)
    def _emit_so():
        so_ref[...] = so_acc[...]


def _layer3_body(r_ref, hcat_ref, hs_ref, sh_ref, deg_ref, add_ref, rs_ref,
                 w_ref, b_ref, wf1_ref, bf1_ref, wf2_ref, bf2_ref,
                 out_ref, p_acc):
    i = pl.program_id(0)
    h_self = hs_ref[...]

    @pl.when(i == 0)
    def _init():
        p_acc[...] = jnp.zeros_like(p_acc)

    dot = jnp.dot(r_ref[...], hcat_ref[...], preferred_element_type=F32)
    agg = dot[:, :HID] + dot[:, HID:]
    agg = agg + (0.5 + rs_ref[...] * (1.0 / N)) * sh_ref[...] \
        + add_ref[...] * h_self
    h3 = _sage_update(h_self, agg / deg_ref[...], w_ref[...], b_ref[...], HID)

    p_acc[...] += jnp.sum(jnp.where(_row_mask(i, TI2), h3, 0.0), axis=0,
                          keepdims=True)

    @pl.when(i == pl.num_programs(0) - 1)
    def _head():
        p = p_acc[...]
        f = _tanh(_dot_pair(p, wf1_ref[...], 2 * HID) + bf1_ref[...])
        out_ref[...] = _dot_pair(f, wf2_ref[...], 1) + bf2_ref[...]


def _row_spec(ti, width):
    return pl.BlockSpec((ti, width), lambda i: (i, 0))


def _full_spec(shape):
    nd = len(shape)
    return pl.BlockSpec(shape, lambda i: (0,) * nd)


def kernel(x, a, W1, b1, W2, b2, W3, b3, Wf1, bf1, Wf2, bf2):
    x = x.astype(F32)
    a = a.astype(F32)

    x_hi = jax.lax.optimization_barrier(x.astype(BF16))
    x_lo = (x - x_hi.astype(F32)).astype(BF16)
    xcat = jnp.concatenate([x_hi, x_lo], axis=1)
    def wpack(w):
        hi = jax.lax.optimization_barrier(w.astype(BF16))
        lo = (w - hi.astype(F32)).astype(BF16)
        return jnp.concatenate([hi, lo], axis=1)

    W1c, W2c, W3c = wpack(W1), wpack(W2), wpack(W3)
    Wf1c, Wf2c = wpack(Wf1), wpack(Wf2)
    b1r = b1.reshape(1, HID)
    b2r = b2.reshape(1, HID)
    b3r = b3.reshape(1, HID)
    bf1r = bf1.reshape(1, 2 * HID)
    bf2r = bf2.reshape(1, 1)

    grid1 = ((N + TI1 - 1) // TI1,)
    grid2 = ((N + TI2 - 1) // TI2,)

    h1, h1cat, sh1, deg, add, r16, rs = pl.pallas_call(
        _layer1_body,
        grid=grid1,
        in_specs=[
            _row_spec(TI1, N),
            pl.BlockSpec((TI1, TI1), lambda i: (i, i)),
            _full_spec((N, 2 * D)),
            _row_spec(TI1, D),
            _full_spec((2 * D, 2 * HID)),
            _full_spec((1, HID)),
        ],
        out_specs=[_row_spec(TI1, HID), _row_spec(TI1, 2 * HID),
                   pl.BlockSpec((1, HID), lambda i: (0, 0)),
                   _row_spec(TI1, 1), _row_spec(TI1, 1), _row_spec(TI1, N),
                   _row_spec(TI1, 1)],
        out_shape=[
            jax.ShapeDtypeStruct((N, HID), F32),
            jax.ShapeDtypeStruct((N, 2 * HID), BF16),
            jax.ShapeDtypeStruct((1, HID), F32),
            jax.ShapeDtypeStruct((N, 1), F32),
            jax.ShapeDtypeStruct((N, 1), F32),
            jax.ShapeDtypeStruct((N, N), BF16),
            jax.ShapeDtypeStruct((N, 1), F32),
        ],
        scratch_shapes=[pltpu.VMEM((1, D), F32), pltpu.VMEM((1, HID), F32)],
        compiler_params=_PARAMS,
    )(a, a, xcat, x, W1c, b1r)

    h2, h2cat, sh2 = pl.pallas_call(
        _layer2_body,
        grid=grid2,
        in_specs=[
            _row_spec(TI2, N),
            _full_spec((N, 2 * HID)),
            _row_spec(TI2, HID),
            _full_spec((1, HID)),
            _row_spec(TI2, 1),
            _row_spec(TI2, 1),
            _row_spec(TI2, 1),
            _full_spec((2 * HID, 2 * HID)),
            _full_spec((1, HID)),
        ],
        out_specs=[_row_spec(TI2, HID), _row_spec(TI2, 2 * HID),
                   pl.BlockSpec((1, HID), lambda i: (0, 0))],
        out_shape=[
            jax.ShapeDtypeStruct((N, HID), F32),
            jax.ShapeDtypeStruct((N, 2 * HID), BF16),
            jax.ShapeDtypeStruct((1, HID), F32),
        ],
        scratch_shapes=[pltpu.VMEM((1, HID), F32)],
        compiler_params=_PARAMS,
    )(r16, h1cat, h1, sh1, deg, add, rs, W2c, b2r)

    out = pl.pallas_call(
        _layer3_body,
        grid=grid2,
        in_specs=[
            _row_spec(TI2, N),
            _full_spec((N, 2 * HID)),
            _row_spec(TI2, HID),
            _full_spec((1, HID)),
            _row_spec(TI2, 1),
            _row_spec(TI2, 1),
            _row_spec(TI2, 1),
            _full_spec((2 * HID, 2 * HID)),
            _full_spec((1, HID)),
            _full_spec((HID, 4 * HID)),
            _full_spec((1, 2 * HID)),
            _full_spec((2 * HID, 2)),
            _full_spec((1, 1)),
        ],
        out_specs=pl.BlockSpec((1, 1), lambda i: (0, 0)),
        out_shape=jax.ShapeDtypeStruct((1, 1), F32),
        scratch_shapes=[pltpu.VMEM((1, HID), F32)],
        compiler_params=_PARAMS,
    )(r16, h2cat, h2, sh2, deg, add, rs, W3c, b3r, Wf1c, bf1r, Wf2c, bf2r)

    return out


# speedup vs baseline: 1.2131x; 1.2131x over previous
import jax
import jax.numpy as jnp
from jax.experimental import pallas as pl
from jax.experimental.pallas import tpu as pltpu

N = 10000
D = 128
HID = 32
TOL = 1e-5
TI1 = 256
TI2 = 512
F32 = jnp.float32
BF16 = jnp.bfloat16
_PARAMS = pltpu.CompilerParams(vmem_limit_bytes=63 * 1024 * 1024)


def _self_loop_add(diag_blk, ti):
    rows = jax.lax.broadcasted_iota(jnp.int32, (ti, ti), 0)
    cols = jax.lax.broadcasted_iota(jnp.int32, (ti, ti), 1)
    dvals = jnp.sum(jnp.where(rows == cols, diag_blk, 0.0), axis=1,
                    keepdims=True)
    return jnp.where(jnp.abs(dvals) < TOL, 1.0, 0.0)


def _row_mask(i, ti):
    row_ids = i * ti + jax.lax.broadcasted_iota(jnp.int32, (ti, 1), 0)
    return row_ids < N


def _tanh(x):
    a1 = 4.89352455891786e-03
    a3 = 6.37261928875436e-04
    a5 = 1.48572235717979e-05
    a7 = 5.12229709037114e-08
    a9 = -8.60467152213735e-11
    a11 = 2.00018790482477e-13
    a13 = -2.76076847742355e-16
    b0 = 4.89352518554385e-03
    b2 = 2.26843463243900e-03
    b4 = 1.18534705686654e-04
    b6 = 1.19825839466702e-06
    t = jnp.clip(x, -7.90531110763549805, 7.90531110763549805)
    s = t * t
    p = a11 + s * a13
    p = a9 + s * p
    p = a7 + s * p
    p = a5 + s * p
    p = a3 + s * p
    p = a1 + s * p
    p = t * p
    q = b4 + s * b6
    q = b2 + s * q
    q = b0 + s * q
    y = p / q
    return jnp.where(jnp.abs(x) < 0.0004, x, y)


def _rsqrt(m):
    s = jax.lax.rsqrt(m)
    s = s * (1.5 - 0.5 * m * s * s)
    s = s * (1.5 - 0.5 * m * s * s)
    return s


def _dot1(u, w):
    return jnp.dot(u.astype(BF16), w, preferred_element_type=F32)


def _sage_update(h_self, aggm, w16, b):
    u = jnp.concatenate([h_self, aggm], axis=1)
    h = _dot1(u, w16) + b
    sq = jnp.sum(h * h, axis=-1, keepdims=True)
    h = h * _rsqrt(jnp.maximum(sq, 1e-12))
    return _tanh(h)


def _layer1_body(a_ref, adiag_ref, x16_ref, xs_ref, w_ref, b_ref,
                 h_ref, h16_ref, deg_ref, add_ref, ahi_ref):
    x_self = xs_ref[...]
    add = _self_loop_add(adiag_ref[...], TI1)

    a_blk = a_ref[...]
    a_hi = a_blk.astype(BF16)
    ahi_ref[...] = a_hi

    agg = jnp.dot(a_hi, x16_ref[...], preferred_element_type=F32)
    deg = jnp.sum(a_blk, axis=1, keepdims=True) + add
    agg = agg + add * x_self
    h = _sage_update(x_self, agg / deg, w_ref[...], b_ref[...])

    h_ref[...] = h
    h16_ref[...] = h.astype(BF16)
    deg_ref[...] = deg
    add_ref[...] = add


def _layer2_body(ahi_ref, h16_ref, hs_ref, deg_ref, add_ref,
                 w_ref, b_ref, o_ref, o16_ref):
    h_self = hs_ref[...]
    agg = jnp.dot(ahi_ref[...], h16_ref[...], preferred_element_type=F32)
    agg = agg + add_ref[...] * h_self
    h = _sage_update(h_self, agg / deg_ref[...], w_ref[...], b_ref[...])
    o_ref[...] = h
    o16_ref[...] = h.astype(BF16)


def _layer3_body(ahi_ref, h16_ref, hs_ref, deg_ref, add_ref,
                 w_ref, b_ref, wf1_ref, bf1_ref, wf2_ref, bf2_ref,
                 out_ref, p_acc):
    i = pl.program_id(0)
    h_self = hs_ref[...]

    @pl.when(i == 0)
    def _init():
        p_acc[...] = jnp.zeros_like(p_acc)

    agg = jnp.dot(ahi_ref[...], h16_ref[...], preferred_element_type=F32)
    agg = agg + add_ref[...] * h_self
    h3 = _sage_update(h_self, agg / deg_ref[...], w_ref[...], b_ref[...])

    p_acc[...] += jnp.sum(jnp.where(_row_mask(i, TI2), h3, 0.0), axis=0,
                          keepdims=True)

    @pl.when(i == pl.num_programs(0) - 1)
    def _head():
        p = p_acc[...]
        f = _tanh(_dot1(p, wf1_ref[...]) + bf1_ref[...])
        out_ref[...] = _dot1(f, wf2_ref[...]) + bf2_ref[...]


def _row_spec(ti, width):
    return pl.BlockSpec((ti, width), lambda i: (i, 0))


def _full_spec(shape):
    nd = len(shape)
    return pl.BlockSpec(shape, lambda i: (0,) * nd)


def kernel(x, a, W1, b1, W2, b2, W3, b3, Wf1, bf1, Wf2, bf2):
    x = x.astype(F32)
    a = a.astype(F32)

    x16 = x.astype(BF16)
    W1c = W1.astype(BF16)
    W2c = W2.astype(BF16)
    W3c = W3.astype(BF16)
    Wf1c = Wf1.astype(BF16)
    Wf2c = Wf2.astype(BF16)
    b1r = b1.reshape(1, HID)
    b2r = b2.reshape(1, HID)
    b3r = b3.reshape(1, HID)
    bf1r = bf1.reshape(1, 2 * HID)
    bf2r = bf2.reshape(1, 1)

    grid1 = ((N + TI1 - 1) // TI1,)
    grid2 = ((N + TI2 - 1) // TI2,)

    h1, h116, deg, add, ahi = pl.pallas_call(
        _layer1_body,
        grid=grid1,
        in_specs=[
            _row_spec(TI1, N),
            pl.BlockSpec((TI1, TI1), lambda i: (i, i)),
            _full_spec((N, D)),
            _row_spec(TI1, D),
            _full_spec((2 * D, HID)),
            _full_spec((1, HID)),
        ],
        out_specs=[_row_spec(TI1, HID), _row_spec(TI1, HID),
                   _row_spec(TI1, 1), _row_spec(TI1, 1),
                   _row_spec(TI1, N)],
        out_shape=[
            jax.ShapeDtypeStruct((N, HID), F32),
            jax.ShapeDtypeStruct((N, HID), BF16),
            jax.ShapeDtypeStruct((N, 1), F32),
            jax.ShapeDtypeStruct((N, 1), F32),
            jax.ShapeDtypeStruct((N, N), BF16),
        ],
        compiler_params=_PARAMS,
    )(a, a, x16, x, W1c, b1r)

    h2, h216 = pl.pallas_call(
        _layer2_body,
        grid=grid2,
        in_specs=[
            _row_spec(TI2, N),
            _full_spec((N, HID)),
            _row_spec(TI2, HID),
            _row_spec(TI2, 1),
            _row_spec(TI2, 1),
            _full_spec((2 * HID, HID)),
            _full_spec((1, HID)),
        ],
        out_specs=[_row_spec(TI2, HID), _row_spec(TI2, HID)],
        out_shape=[
            jax.ShapeDtypeStruct((N, HID), F32),
            jax.ShapeDtypeStruct((N, HID), BF16),
        ],
        compiler_params=_PARAMS,
    )(ahi, h116, h1, deg, add, W2c, b2r)

    out = pl.pallas_call(
        _layer3_body,
        grid=grid2,
        in_specs=[
            _row_spec(TI2, N),
            _full_spec((N, HID)),
            _row_spec(TI2, HID),
            _row_spec(TI2, 1),
            _row_spec(TI2, 1),
            _full_spec((2 * HID, HID)),
            _full_spec((1, HID)),
            _full_spec((HID, 2 * HID)),
            _full_spec((1, 2 * HID)),
            _full_spec((2 * HID, 1)),
            _full_spec((1, 1)),
        ],
        out_specs=pl.BlockSpec((1, 1), lambda i: (0, 0)),
        out_shape=jax.ShapeDtypeStruct((1, 1), F32),
        scratch_shapes=[pltpu.VMEM((1, HID), F32)],
        compiler_params=_PARAMS,
    )(ahi, h216, h2, deg, add, W3c, b3r, Wf1c, bf1r, Wf2c, bf2r)

    return out
